# baseline (device time: 12942 ns/iter reference)
import jax
import jax.numpy as jnp
from jax import lax
from jax.experimental import pallas as pl
from jax.experimental.pallas import tpu as pltpu

N_DEV = 4
N_CHUNKS = 8


def kernel(x):
    m, n = x.shape
    block_m = m // N_CHUNKS

    def body(
        x_hbm,
        out_hbm,
        chunk_buf,
        comm_ref,
        out_vmem,
        copy_sems,
        send_sems,
        recv_sems,
        out_sem,
    ):
        my = lax.axis_index("i")
        barrier_sem = pltpu.get_barrier_semaphore()

        for off in range(1, N_DEV):
            pl.semaphore_signal(
                barrier_sem,
                inc=1,
                device_id=((my + off) % N_DEV,),
                device_id_type=pl.DeviceIdType.MESH,
            )

        copies = [
            pltpu.make_async_copy(
                x_hbm.at[pl.ds(c * block_m, block_m), :],
                chunk_buf.at[c % 2],
                copy_sems.at[c % 2],
            )
            for c in range(N_CHUNKS)
        ]
        copies[0].start()
        acc = None
        for c in range(N_CHUNKS):
            if c + 1 < N_CHUNKS:
                copies[c + 1].start()
            copies[c].wait()
            part = jnp.sum(chunk_buf[c % 2], axis=0, keepdims=True)
            acc = part if acc is None else acc + part

        comm_ref[N_DEV - 1] = acc
        pl.semaphore_wait(barrier_sem, N_DEV - 1)

        rdmas = []
        for off in range(1, N_DEV):
            rdma = pltpu.make_async_remote_copy(
                src_ref=comm_ref.at[N_DEV - 1],
                dst_ref=comm_ref.at[off - 1],
                send_sem=send_sems.at[off - 1],
                recv_sem=recv_sems.at[off - 1],
                device_id=((my + off) % N_DEV,),
                device_id_type=pl.DeviceIdType.MESH,
            )
            rdma.start()
            rdmas.append(rdma)
        for rdma in rdmas:
            rdma.wait()

        out_vmem[:, :] = comm_ref[0] + comm_ref[1] + comm_ref[2] + comm_ref[3]
        out_copy = pltpu.make_async_copy(out_vmem, out_hbm, out_sem)
        out_copy.start()
        out_copy.wait()

    return pl.pallas_call(
        body,
        out_shape=jax.ShapeDtypeStruct((1, n), x.dtype),
        in_specs=[pl.BlockSpec(memory_space=pl.ANY)],
        out_specs=pl.BlockSpec(memory_space=pl.ANY),
        scratch_shapes=[
            pltpu.VMEM((2, block_m, n), x.dtype),
            pltpu.VMEM((N_DEV, 1, n), x.dtype),
            pltpu.VMEM((1, n), x.dtype),
            pltpu.SemaphoreType.DMA((2,)),
            pltpu.SemaphoreType.DMA((N_DEV - 1,)),
            pltpu.SemaphoreType.DMA((N_DEV - 1,)),
            pltpu.SemaphoreType.DMA,
        ],
        compiler_params=pltpu.CompilerParams(collective_id=0),
    )(x)


# device time: 12874 ns/iter; 1.0053x vs baseline; 1.0053x over previous
import jax
import jax.numpy as jnp
from jax import lax
from jax.experimental import pallas as pl
from jax.experimental.pallas import tpu as pltpu

N_DEV = 4
N_CHUNKS = 8


def kernel(x):
    m, n = x.shape
    block_m = m // N_CHUNKS

    def body(
        x_hbm,
        out_hbm,
        chunk_buf,
        comm_ref,
        out_vmem,
        copy_sems,
        send_sems,
        recv_sems,
        out_sem,
    ):
        my = lax.axis_index("i")
        barrier_sem = pltpu.get_barrier_semaphore()

        for off in range(1, N_DEV):
            pl.semaphore_signal(
                barrier_sem,
                inc=1,
                device_id=((my + off) % N_DEV,),
                device_id_type=pl.DeviceIdType.MESH,
            )

        copies = [
            pltpu.make_async_copy(
                x_hbm.at[pl.ds(c * block_m, block_m), :],
                chunk_buf.at[c % 2],
                copy_sems.at[c % 2],
            )
            for c in range(N_CHUNKS)
        ]
        copies[0].start()
        acc = None
        for c in range(N_CHUNKS):
            if c + 1 < N_CHUNKS:
                copies[c + 1].start()
            copies[c].wait()
            part = jnp.sum(chunk_buf[c % 2], axis=0, keepdims=True)
            acc = part if acc is None else acc + part

        comm_ref[N_DEV - 1] = acc
        pl.semaphore_wait(barrier_sem, N_DEV - 1)

        rdmas = []
        for off in range(1, N_DEV):
            rdma = pltpu.make_async_remote_copy(
                src_ref=comm_ref.at[N_DEV - 1],
                dst_ref=comm_ref.at[off - 1],
                send_sem=send_sems.at[off - 1],
                recv_sem=recv_sems.at[off - 1],
                device_id=((my + off) % N_DEV,),
                device_id_type=pl.DeviceIdType.MESH,
            )
            rdma.start()
            rdmas.append(rdma)
        for rdma in rdmas:
            rdma.wait()

        out_vmem[:, :] = comm_ref[0] + comm_ref[1] + comm_ref[2] + comm_ref[3]
        out_copy = pltpu.make_async_copy(out_vmem, out_hbm, out_sem)
        out_copy.start()
        out_copy.wait()

    x = pltpu.with_memory_space_constraint(x, pltpu.MemorySpace.HBM)

    return pl.pallas_call(
        body,
        out_shape=jax.ShapeDtypeStruct((1, n), x.dtype),
        in_specs=[pl.BlockSpec(memory_space=pl.ANY)],
        out_specs=pl.BlockSpec(memory_space=pl.ANY),
        scratch_shapes=[
            pltpu.VMEM((2, block_m, n), x.dtype),
            pltpu.VMEM((N_DEV, 1, n), x.dtype),
            pltpu.VMEM((1, n), x.dtype),
            pltpu.SemaphoreType.DMA((2,)),
            pltpu.SemaphoreType.DMA((N_DEV - 1,)),
            pltpu.SemaphoreType.DMA((N_DEV - 1,)),
            pltpu.SemaphoreType.DMA,
        ],
        compiler_params=pltpu.CompilerParams(collective_id=0),
    )(x)


# device time: 11093 ns/iter; 1.1667x vs baseline; 1.1606x over previous
import jax
import jax.numpy as jnp
from jax import lax
from jax.experimental import pallas as pl
from jax.experimental.pallas import tpu as pltpu

N_DEV = 4
N_CHUNKS = 8
N_BUF = 4


def kernel(x):
    m, n = x.shape
    block_m = m // N_CHUNKS

    def body(
        x_hbm,
        out_hbm,
        chunk_buf,
        comm_ref,
        out_vmem,
        copy_sems,
        send_sems,
        recv_sems,
        out_sem,
    ):
        my = lax.axis_index("i")
        barrier_sem = pltpu.get_barrier_semaphore()

        for off in range(1, N_DEV):
            pl.semaphore_signal(
                barrier_sem,
                inc=1,
                device_id=((my + off) % N_DEV,),
                device_id_type=pl.DeviceIdType.MESH,
            )

        copies = [
            pltpu.make_async_copy(
                x_hbm.at[pl.ds(c * block_m, block_m), :],
                chunk_buf.at[c % N_BUF],
                copy_sems.at[c % N_BUF],
            )
            for c in range(N_CHUNKS)
        ]
        for c in range(N_BUF - 1):
            copies[c].start()
        acc = None
        for c in range(N_CHUNKS):
            if c + N_BUF - 1 < N_CHUNKS:
                copies[c + N_BUF - 1].start()
            copies[c].wait()
            part = jnp.sum(chunk_buf[c % N_BUF], axis=0, keepdims=True)
            acc = part if acc is None else acc + part

        comm_ref[N_DEV - 1] = acc
        pl.semaphore_wait(barrier_sem, N_DEV - 1)

        rdmas = []
        for off in range(1, N_DEV):
            rdma = pltpu.make_async_remote_copy(
                src_ref=comm_ref.at[N_DEV - 1],
                dst_ref=comm_ref.at[off - 1],
                send_sem=send_sems.at[off - 1],
                recv_sem=recv_sems.at[off - 1],
                device_id=((my + off) % N_DEV,),
                device_id_type=pl.DeviceIdType.MESH,
            )
            rdma.start()
            rdmas.append(rdma)
        for rdma in rdmas:
            rdma.wait()

        out_vmem[:, :] = comm_ref[0] + comm_ref[1] + comm_ref[2] + comm_ref[3]
        out_copy = pltpu.make_async_copy(out_vmem, out_hbm, out_sem)
        out_copy.start()
        out_copy.wait()

    x = pltpu.with_memory_space_constraint(x, pltpu.MemorySpace.HBM)

    return pl.pallas_call(
        body,
        out_shape=jax.ShapeDtypeStruct((1, n), x.dtype),
        in_specs=[pl.BlockSpec(memory_space=pl.ANY)],
        out_specs=pl.BlockSpec(memory_space=pl.ANY),
        scratch_shapes=[
            pltpu.VMEM((N_BUF, block_m, n), x.dtype),
            pltpu.VMEM((N_DEV, 1, n), x.dtype),
            pltpu.VMEM((1, n), x.dtype),
            pltpu.SemaphoreType.DMA((N_BUF,)),
            pltpu.SemaphoreType.DMA((N_DEV - 1,)),
            pltpu.SemaphoreType.DMA((N_DEV - 1,)),
            pltpu.SemaphoreType.DMA,
        ],
        compiler_params=pltpu.CompilerParams(collective_id=0),
    )(x)
